# Initial kernel scaffold; baseline (speedup 1.0000x reference)
import jax
import jax.numpy as jnp
from jax.experimental import pallas as pl


def kernel(x, cluster_centers, filling_target, means_target, covs_target):
    raise NotImplementedError("write your pallas kernel here")


# speedup vs baseline: 2.0346x; 2.0346x over previous
import functools

import jax
import jax.numpy as jnp
from jax.experimental import pallas as pl
from jax.experimental.pallas import tpu as pltpu

N, K, D = 16384, 64, 64
BLK = 256


def _loss_kernel(xt_ref, x_ref, c_ref, ft_ref, mt_ref, ct_ref, out_ref,
                 counts_acc, sums_acc, s_acc, *, nblk):
    i = pl.program_id(0)

    @pl.when(i == 0)
    def _init():
        counts_acc[:, :] = jnp.zeros_like(counts_acc)
        sums_acc[:, :] = jnp.zeros_like(sums_acc)
        s_acc[:, :] = jnp.zeros_like(s_acc)

    xt = xt_ref[:, :]
    xb = x_ref[:, :]
    c = c_ref[:, :]

    cn = jnp.sum(c * c, axis=1, keepdims=True)
    xn = jnp.sum(xt * xt, axis=0, keepdims=True)
    d2 = cn - 2.0 * jnp.dot(c, xt, preferred_element_type=jnp.float32) + xn

    dmin = jnp.min(d2, axis=0, keepdims=True)
    kio = jax.lax.broadcasted_iota(jnp.int32, (K, BLK), 0)
    pred = jnp.min(jnp.where(d2 <= dmin, kio, K), axis=0, keepdims=True)
    onehot = (kio == pred).astype(jnp.float32)

    counts_acc[:, :] += jnp.sum(onehot, axis=1, keepdims=True)

    oh_bf = onehot.astype(jnp.bfloat16)
    xb_bf = xb.astype(jnp.bfloat16)
    xt_bf = xt.astype(jnp.bfloat16)

    sums_acc[:, :] += jnp.dot(oh_bf, xb_bf,
                              preferred_element_type=jnp.float32)

    m_oh = jnp.reshape(jnp.broadcast_to(oh_bf[:, None, :], (K, D, BLK)),
                       (K * D, BLK))
    m_xt = jnp.reshape(jnp.broadcast_to(xt_bf[None, :, :], (K, D, BLK)),
                       (K * D, BLK))
    s_acc[:, :] += jnp.dot(m_oh * m_xt, xb_bf,
                           preferred_element_type=jnp.float32)

    @pl.when(i == nblk - 1)
    def _epilogue():
        counts = counts_acc[:, :]
        safe = jnp.maximum(counts, 1.0)
        means = sums_acc[:, :] / safe

        filling = counts / jnp.float32(N)
        loss_fil = jnp.sum((filling - ft_ref[:, :]) ** 2,
                           axis=(0, 1), keepdims=True) / jnp.float32(K)
        loss_means = jnp.sum((means - mt_ref[:, :]) ** 2,
                             axis=(0, 1), keepdims=True) / jnp.float32(K * D)

        m3 = jnp.reshape(jnp.broadcast_to(means[:, None, :], (K, D, D)),
                         (K * D, D))
        rio = jax.lax.broadcasted_iota(jnp.int32, (K * D, D), 0)
        jio = jax.lax.broadcasted_iota(jnp.int32, (K * D, D), 1)
        isel = (rio % D == jio).astype(jnp.float32)
        m4 = jnp.sum(m3 * isel, axis=1, keepdims=True)

        countsb = jnp.reshape(jnp.broadcast_to(counts[:, :, None], (K, D, 1)),
                              (K * D, 1))
        denomb = jnp.maximum(countsb - 1.0, 1.0)
        covs = (s_acc[:, :] - countsb * (m4 * m3)) / denomb
        loss_covs = jnp.sum((covs - ct_ref[:, :]) ** 2,
                            axis=(0, 1), keepdims=True) / jnp.float32(K * D * D)

        out_ref[:, :] = loss_fil + loss_means + loss_covs


def kernel(x, cluster_centers, filling_target, means_target, covs_target):
    nblk = N // BLK
    xt = x.T
    ft = filling_target.reshape(K, 1)
    ct = covs_target.reshape(K * D, D)

    out = pl.pallas_call(
        functools.partial(_loss_kernel, nblk=nblk),
        grid=(nblk,),
        in_specs=[
            pl.BlockSpec((D, BLK), lambda i: (0, i)),
            pl.BlockSpec((BLK, D), lambda i: (i, 0)),
            pl.BlockSpec((K, D), lambda i: (0, 0)),
            pl.BlockSpec((K, 1), lambda i: (0, 0)),
            pl.BlockSpec((K, D), lambda i: (0, 0)),
            pl.BlockSpec((K * D, D), lambda i: (0, 0)),
        ],
        out_specs=pl.BlockSpec((1, 1), lambda i: (0, 0)),
        out_shape=jax.ShapeDtypeStruct((1, 1), jnp.float32),
        scratch_shapes=[
            pltpu.VMEM((K, 1), jnp.float32),
            pltpu.VMEM((K, D), jnp.float32),
            pltpu.VMEM((K * D, D), jnp.float32),
        ],
        compiler_params=pltpu.CompilerParams(
            dimension_semantics=("arbitrary",),
        ),
    )(xt, x, cluster_centers, ft, means_target, ct)
    return out[0, 0]


# speedup vs baseline: 2.4076x; 1.1833x over previous
import functools

import jax
import jax.numpy as jnp
from jax.experimental import pallas as pl
from jax.experimental.pallas import tpu as pltpu

N, K, D = 16384, 64, 64
BLK = 512


def _loss_kernel(xt_ref, x_ref, c_ref, ft_ref, mt_ref, ct_ref, out_ref,
                 counts_acc, sums_acc, s_acc, *, nblk):
    i = pl.program_id(0)

    @pl.when(i == 0)
    def _init():
        counts_acc[:, :] = jnp.zeros_like(counts_acc)
        sums_acc[:, :] = jnp.zeros_like(sums_acc)
        s_acc[:, :] = jnp.zeros_like(s_acc)

    xt = xt_ref[:, :]
    xb = x_ref[:, :]
    c = c_ref[:, :]

    cn = jnp.sum(c * c, axis=1, keepdims=True)
    xn = jnp.sum(xt * xt, axis=0, keepdims=True)
    d2 = cn - 2.0 * jnp.dot(c, xt, preferred_element_type=jnp.float32) + xn

    dmin = jnp.min(d2, axis=0, keepdims=True)
    kio = jax.lax.broadcasted_iota(jnp.int32, (K, BLK), 0)
    pred = jnp.min(jnp.where(d2 <= dmin, kio, K), axis=0, keepdims=True)
    onehot = (kio == pred).astype(jnp.float32)

    counts_acc[:, :] += jnp.sum(onehot, axis=1, keepdims=True)

    oh_bf = onehot.astype(jnp.bfloat16)
    xb_bf = xb.astype(jnp.bfloat16)
    xt_bf = xt.astype(jnp.bfloat16)

    sums_acc[:, :] += jnp.dot(oh_bf, xb_bf,
                              preferred_element_type=jnp.float32)

    m_oh = jnp.reshape(jnp.broadcast_to(oh_bf[:, None, :], (K, D, BLK)),
                       (K * D, BLK))
    m_xt = jnp.reshape(jnp.broadcast_to(xt_bf[None, :, :], (K, D, BLK)),
                       (K * D, BLK))
    s_acc[:, :] += jnp.dot(m_oh * m_xt, xb_bf,
                           preferred_element_type=jnp.float32)

    @pl.when(i == nblk - 1)
    def _epilogue():
        counts = counts_acc[:, :]
        safe = jnp.maximum(counts, 1.0)
        means = sums_acc[:, :] / safe

        filling = counts / jnp.float32(N)
        loss_fil = jnp.sum((filling - ft_ref[:, :]) ** 2,
                           axis=(0, 1), keepdims=True) / jnp.float32(K)
        loss_means = jnp.sum((means - mt_ref[:, :]) ** 2,
                             axis=(0, 1), keepdims=True) / jnp.float32(K * D)

        m3 = jnp.reshape(jnp.broadcast_to(means[:, None, :], (K, D, D)),
                         (K * D, D))
        rio = jax.lax.broadcasted_iota(jnp.int32, (K * D, D), 0)
        jio = jax.lax.broadcasted_iota(jnp.int32, (K * D, D), 1)
        isel = (rio % D == jio).astype(jnp.float32)
        m4 = jnp.sum(m3 * isel, axis=1, keepdims=True)

        countsb = jnp.reshape(jnp.broadcast_to(counts[:, :, None], (K, D, 1)),
                              (K * D, 1))
        denomb = jnp.maximum(countsb - 1.0, 1.0)
        covs = (s_acc[:, :] - countsb * (m4 * m3)) / denomb
        loss_covs = jnp.sum((covs - ct_ref[:, :]) ** 2,
                            axis=(0, 1), keepdims=True) / jnp.float32(K * D * D)

        out_ref[:, :] = loss_fil + loss_means + loss_covs


def kernel(x, cluster_centers, filling_target, means_target, covs_target):
    nblk = N // BLK
    xt = x.T
    ft = filling_target.reshape(K, 1)
    ct = covs_target.reshape(K * D, D)

    out = pl.pallas_call(
        functools.partial(_loss_kernel, nblk=nblk),
        grid=(nblk,),
        in_specs=[
            pl.BlockSpec((D, BLK), lambda i: (0, i)),
            pl.BlockSpec((BLK, D), lambda i: (i, 0)),
            pl.BlockSpec((K, D), lambda i: (0, 0)),
            pl.BlockSpec((K, 1), lambda i: (0, 0)),
            pl.BlockSpec((K, D), lambda i: (0, 0)),
            pl.BlockSpec((K * D, D), lambda i: (0, 0)),
        ],
        out_specs=pl.BlockSpec((1, 1), lambda i: (0, 0)),
        out_shape=jax.ShapeDtypeStruct((1, 1), jnp.float32),
        scratch_shapes=[
            pltpu.VMEM((K, 1), jnp.float32),
            pltpu.VMEM((K, D), jnp.float32),
            pltpu.VMEM((K * D, D), jnp.float32),
        ],
        compiler_params=pltpu.CompilerParams(
            dimension_semantics=("arbitrary",),
        ),
    )(xt, x, cluster_centers, ft, means_target, ct)
    return out[0, 0]
